# baseline (speedup 1.0000x reference)
import jax
import jax.numpy as jnp
from jax import lax
from jax.experimental import pallas as pl
from jax.experimental.pallas import tpu as pltpu
from jax.experimental.pallas import tpu_sc as plsc

_K = 256
_RB = 8
_V = 100000
_CHUNK = 10000
_NCHUNK = _V // _CHUNK
_NTILES = 32
_RPT = 128 // _NTILES
_L = 16


def _sortable(b):
    return b ^ ((b >> 31) & jnp.int32(0x7FFFFFFF))


def _make_sc_body(rpt):
  def _sc_body(pb_hbm, pd_hbm, lik_hbm, out_hbm,
             l0, l1, b0, b1, d0, d1, hist_b, hist_d, obuf,
             sl0, sl1, sb0, sb1, sd0, sd1):
    wid = lax.axis_index("s") * 2 + lax.axis_index("c")
    lane = lax.iota(jnp.int32, _L)
    ones = jnp.ones((_L,), jnp.int32)
    zero = jnp.zeros((_L,), jnp.int32)
    slots = ((l0, b0, d0), (l1, b1, d1))
    sems = ((sl0, sb0, sd0), (sl1, sb1, sd1))

    def start(row, c, slot):
        off = row * _V + c * _CHUNK
        lr, br, dr = slots[slot]
        ls, bs, ds_ = sems[slot]
        return (
            pltpu.async_copy(lik_hbm.at[pl.ds(off, _CHUNK)], lr, ls),
            pltpu.async_copy(pb_hbm.at[pl.ds(off, _CHUNK)], br, bs),
            pltpu.async_copy(pd_hbm.at[pl.ds(off, _CHUNK)], dr, ds_),
        )

    def zero_hists():
        @plsc.parallel_loop(0, 4096 // _L, 1, unroll=8)
        def _(j):
            hist_b[pl.ds(j * _L, _L)] = zero
            hist_d[pl.ds(j * _L, _L)] = zero

    def chunk_loop(slot, fn, carry):
        lr, br, dr = slots[slot]

        def body(i, car):
            sl = pl.ds(i * _L, _L)
            l = lr[sl]
            xb = br[sl] + l
            xd = dr[sl] + l
            sb = _sortable(lax.bitcast_convert_type(xb, jnp.int32))
            sd = _sortable(lax.bitcast_convert_type(xd, jnp.int32))
            return fn(xb, xd, sb, sd, car)

        return plsc.parallel_loop(0, _CHUNK // _L, 1, unroll=5,
                                  carry=carry)(body)

    def stream_pass(row, fn, carry):
        hs = start(row, 0, 0)
        for c in range(_NCHUNK):
            nxt = start(row, c + 1, (c + 1) & 1) if c + 1 < _NCHUNK else None
            for h in hs:
                h.wait()
            carry = chunk_loop(c & 1, fn, carry)
            hs = nxt
        return carry

    def scan_hist(hist_ref, nvregs, running0):
        def body(j, car):
            running, p, c_hi, found = car
            base = (nvregs - 1 - j) * _L
            v = hist_ref[pl.ds(base, _L)]
            rv = lax.rev(v, (0,))
            cs = plsc.cumsum(rv)
            tot = running + cs
            pred = tot >= _K
            npred = plsc.all_reduce_population_count(pred)
            f = plsc.all_reduce_ffs(pred)
            hit = (npred > 0) & jnp.logical_not(found)
            p_new = base + 15 - f
            c_hi_new = running + jnp.sum(jnp.where(lane < f, rv, zero))
            run_next = running + jnp.sum(v)
            return (run_next,
                    jnp.where(hit, p_new, p),
                    jnp.where(hit, c_hi_new, c_hi),
                    found | (npred > 0))

        init = (running0, zero, zero, jnp.zeros((_L,), jnp.bool_))
        _, p, c_hi, _ = plsc.parallel_loop(0, nvregs, 1, unroll=4,
                                           carry=init)(body)
        return p, c_hi

    def row_task(rr, c0):
        row = wid * rpt + rr

        zero_hists()

        def fn0(xb, xd, sb, sd, car):
            mb, md = car
            plsc.addupdate_scatter(hist_b, [(sb >> 20) + 2048], ones)
            plsc.addupdate_scatter(hist_d, [(sd >> 20) + 2048], ones)
            return (jnp.maximum(mb, xb), jnp.maximum(md, xd))

        ninf = jnp.full((_L,), -jnp.inf, jnp.float32)
        mb, md = stream_pass(row, fn0, (ninf, ninf))
        p0b, chib = scan_hist(hist_b, 4096 // _L, zero)
        p0d, chid = scan_hist(hist_d, 4096 // _L, zero)

        zero_hists()

        def fn1(xb, xd, sb, sd, car):
            plsc.addupdate_scatter(hist_b, [(sb >> 8) & 0xFFF], ones,
                                   mask=((sb >> 20) + 2048) == p0b)
            plsc.addupdate_scatter(hist_d, [(sd >> 8) & 0xFFF], ones,
                                   mask=((sd >> 20) + 2048) == p0d)
            return car

        stream_pass(row, fn1, jnp.int32(0))
        p1b, chib = scan_hist(hist_b, 4096 // _L, chib)
        p1d, chid = scan_hist(hist_d, 4096 // _L, chid)

        zero_hists()
        pre_b = ((p0b - 2048) << 12) + p1b
        pre_d = ((p0d - 2048) << 12) + p1d

        def fn2(xb, xd, sb, sd, car):
            plsc.addupdate_scatter(hist_b, [sb & 0xFF], ones,
                                   mask=(sb >> 8) == pre_b)
            plsc.addupdate_scatter(hist_d, [sd & 0xFF], ones,
                                   mask=(sd >> 8) == pre_d)
            return car

        stream_pass(row, fn2, jnp.int32(0))
        p2b, _ = scan_hist(hist_b, 256 // _L, chib)
        p2d, _ = scan_hist(hist_d, 256 // _L, chid)

        s_b = ((p0b - 2048) << 20) + (p1b << 8) + p2b
        s_d = ((p0d - 2048) << 20) + (p1d << 8) + p2d
        tfb = lax.bitcast_convert_type(_sortable(s_b), jnp.float32)
        tfd = lax.bitcast_convert_type(_sortable(s_d), jnp.float32)
        mfb = jnp.zeros((_L,), jnp.float32) + jnp.max(mb)
        mfd = jnp.zeros((_L,), jnp.float32) + jnp.max(md)

        cur = obuf[...]
        cur = jnp.where(lane == rr, tfb, cur)
        cur = jnp.where(lane == rpt + rr, tfd, cur)
        cur = jnp.where(lane == 2 * rpt + rr, mfb, cur)
        cur = jnp.where(lane == 3 * rpt + rr, mfd, cur)
        obuf[...] = cur
        return c0

    lax.fori_loop(0, rpt, row_task, 0)
    pltpu.sync_copy(obuf, out_hbm.at[wid])
  return _sc_body


def _sc_stats(pb, pd, lik, nrows):
    mesh = plsc.VectorSubcoreMesh(core_axis_name="c", subcore_axis_name="s",
                                  num_cores=2, num_subcores=16)
    f32 = jnp.float32
    return pl.kernel(
        _make_sc_body(nrows // _NTILES),
        out_type=jax.ShapeDtypeStruct((_NTILES, _L), f32),
        mesh=mesh,
        compiler_params=pltpu.CompilerParams(needs_layout_passes=False),
        scratch_types=[
            pltpu.VMEM((_CHUNK,), f32), pltpu.VMEM((_CHUNK,), f32),
            pltpu.VMEM((_CHUNK,), f32), pltpu.VMEM((_CHUNK,), f32),
            pltpu.VMEM((_CHUNK,), f32), pltpu.VMEM((_CHUNK,), f32),
            pltpu.VMEM((4096,), jnp.int32), pltpu.VMEM((4096,), jnp.int32),
            pltpu.VMEM((_L,), f32),
            pltpu.SemaphoreType.DMA, pltpu.SemaphoreType.DMA,
            pltpu.SemaphoreType.DMA, pltpu.SemaphoreType.DMA,
            pltpu.SemaphoreType.DMA, pltpu.SemaphoreType.DMA,
        ],
    )(pb, pd, lik)


def _tc_compute(pb_ref, pd_ref, lik_ref, st_ref, out_ref):
    lik = lik_ref[...]
    st = st_ref[...]
    for src, p_ref in ((0, pb_ref), (1, pd_ref)):
        x = p_ref[...] + lik
        t_f = st[:, src:src + 1]
        m_f = st[:, 2 + src:3 + src]
        e = jnp.where(x >= t_f, jnp.exp(x - m_f), jnp.float32(0.0))
        denom = jnp.sum(e, axis=-1, keepdims=True)
        out_ref[src] = e * (jnp.float32(1.0) / denom)


def _tc_body0(pb_ref, pd_ref, lik_ref, st_ref, out_ref):
    _tc_compute(pb_ref, pd_ref, lik_ref, st_ref, out_ref)


def _tc_body1(pb_ref, pd_ref, lik_ref, st_ref, dummy_ref, out_ref):
    del dummy_ref
    _tc_compute(pb_ref, pd_ref, lik_ref, st_ref, out_ref)


def _stats_for_rows(pb, pd, lik, lo, hi):
    nrows = hi - lo
    stats = _sc_stats(pb[lo:hi].reshape(-1), pd[lo:hi].reshape(-1),
                      lik[lo:hi].reshape(-1), nrows)
    rpt = nrows // _NTILES
    st = stats[:, :4 * rpt].reshape(_NTILES, 4, rpt)
    return st.transpose(0, 2, 1).reshape(nrows, 4)


def kernel(prior_bass_logits, prior_drums_logits, likelihood_logits, top_k):
    del top_k
    pb, pd, lik = prior_bass_logits, prior_drums_logits, likelihood_logits
    B, V = pb.shape
    h = B // 2
    hb = h // _RB
    st0 = _stats_for_rows(pb, pd, lik, 0, h)
    st1 = _stats_for_rows(pb, pd, lik, h, B)
    out_shape = jax.ShapeDtypeStruct((2, B, V), jnp.float32)
    in_spec0 = pl.BlockSpec((_RB, V), lambda i: (i, 0))
    st_spec = pl.BlockSpec((_RB, 4), lambda i: (i, 0))
    half0 = pl.pallas_call(
        _tc_body0,
        grid=(hb,),
        in_specs=[in_spec0, in_spec0, in_spec0, st_spec],
        out_specs=pl.BlockSpec((2, _RB, V), lambda i: (0, i, 0)),
        out_shape=out_shape,
    )(pb[:h], pd[:h], lik[:h], st0)
    in_spec1 = pl.BlockSpec((_RB, V), lambda i: (hb + i, 0))
    dummy_spec = pl.BlockSpec((2, _RB, 128), lambda i: (0, 0, 0))
    return pl.pallas_call(
        _tc_body1,
        grid=(hb,),
        in_specs=[in_spec1, in_spec1, in_spec1, st_spec, dummy_spec],
        out_specs=pl.BlockSpec((2, _RB, V), lambda i: (0, hb + i, 0)),
        out_shape=out_shape,
        input_output_aliases={4: 0},
    )(pb, pd, lik, st1, half0)


# speedup vs baseline: 1.1535x; 1.1535x over previous
import jax
import jax.numpy as jnp
from jax import lax
from jax.experimental import pallas as pl
from jax.experimental.pallas import tpu as pltpu
from jax.experimental.pallas import tpu_sc as plsc

_K = 256
_RB = 8
_V = 100000
_CHUNK = 10000
_NCHUNK = _V // _CHUNK
_NTILES = 32
_L = 16
_CAP = 4096


def _sortable(b):
    return b ^ ((b >> 31) & jnp.int32(0x7FFFFFFF))


def _make_sc_body(rpt, collect):
  def _sc_body(pb_hbm, pd_hbm, lik_hbm, out_hbm, flag_hbm,
               l0, l1, b0, b1, d0, d1, hist_b, hist_d, cand_b, cand_d, obuf,
               fbuf, sl0, sl1, sb0, sb1, sd0, sd1):
    wid = lax.axis_index("s") * 2 + lax.axis_index("c")
    lane = lax.iota(jnp.int32, _L)
    ones = jnp.ones((_L,), jnp.int32)
    zero = jnp.zeros((_L,), jnp.int32)
    slots = ((l0, b0, d0), (l1, b1, d1))
    sems = ((sl0, sb0, sd0), (sl1, sb1, sd1))

    def start(row, c, slot):
        off = row * _V + c * _CHUNK
        lr, br, dr = slots[slot]
        ls, bs, ds_ = sems[slot]
        return (
            pltpu.async_copy(lik_hbm.at[pl.ds(off, _CHUNK)], lr, ls),
            pltpu.async_copy(pb_hbm.at[pl.ds(off, _CHUNK)], br, bs),
            pltpu.async_copy(pd_hbm.at[pl.ds(off, _CHUNK)], dr, ds_),
        )

    def zero_hists():
        @plsc.parallel_loop(0, 4096 // _L, 1, unroll=8)
        def _(j):
            hist_b[pl.ds(j * _L, _L)] = zero
            hist_d[pl.ds(j * _L, _L)] = zero

    def chunk_loop(slot, fn, carry):
        lr, br, dr = slots[slot]

        def body(i, car):
            sl = pl.ds(i * _L, _L)
            l = lr[sl]
            xb = br[sl] + l
            xd = dr[sl] + l
            sb = _sortable(lax.bitcast_convert_type(xb, jnp.int32))
            sd = _sortable(lax.bitcast_convert_type(xd, jnp.int32))
            return fn(xb, xd, sb, sd, car)

        return plsc.parallel_loop(0, _CHUNK // _L, 1, unroll=5,
                                  carry=carry)(body)

    def stream_pass(row, fn, carry):
        hs = start(row, 0, 0)
        for c in range(_NCHUNK):
            nxt = start(row, c + 1, (c + 1) & 1) if c + 1 < _NCHUNK else None
            for h in hs:
                h.wait()
            carry = chunk_loop(c & 1, fn, carry)
            hs = nxt
        return carry

    def scan_hist(hist_ref, nvregs, running0):
        def body(j, car):
            running, p, c_hi, found = car
            base = (nvregs - 1 - j) * _L
            v = hist_ref[pl.ds(base, _L)]
            rv = lax.rev(v, (0,))
            cs = plsc.cumsum(rv)
            tot = running + cs
            pred = tot >= _K
            npred = plsc.all_reduce_population_count(pred)
            f = plsc.all_reduce_ffs(pred)
            hit = (npred > 0) & jnp.logical_not(found)
            p_new = base + 15 - f
            c_hi_new = running + jnp.sum(jnp.where(lane < f, rv, zero))
            run_next = running + jnp.sum(v)
            return (run_next,
                    jnp.where(hit, p_new, p),
                    jnp.where(hit, c_hi_new, c_hi),
                    found | (npred > 0))

        init = (running0, zero, zero, jnp.zeros((_L,), jnp.bool_))
        _, p, c_hi, _ = plsc.parallel_loop(0, nvregs, 1, unroll=4,
                                           carry=init)(body)
        return p, c_hi

    def row_task(rr, c0):
        row = wid * rpt + rr

        zero_hists()

        def fn0(xb, xd, sb, sd, car):
            mb, md = car
            plsc.addupdate_scatter(hist_b, [(sb >> 20) + 2048], ones)
            plsc.addupdate_scatter(hist_d, [(sd >> 20) + 2048], ones)
            return (jnp.maximum(mb, xb), jnp.maximum(md, xd))

        ninf = jnp.full((_L,), -jnp.inf, jnp.float32)
        mb, md = stream_pass(row, fn0, (ninf, ninf))
        p0b, chib = scan_hist(hist_b, 4096 // _L, zero)
        p0d, chid = scan_hist(hist_d, 4096 // _L, zero)

        if collect:
            def fnc(xb, xd, sb, sd, car):
                offb, offd = car
                kb = ((sb >> 20) + 2048) == p0b
                kd = ((sd >> 20) + 2048) == p0d
                posb = offb + plsc.cumsum(kb.astype(jnp.int32)) - 1
                posd = offd + plsc.cumsum(kd.astype(jnp.int32)) - 1
                plsc.store_scatter(cand_b, [posb], sb,
                                   mask=kb & (posb < _CAP))
                plsc.store_scatter(cand_d, [posd], sd,
                                   mask=kd & (posd < _CAP))
                return (offb + plsc.all_reduce_population_count(kb),
                        offd + plsc.all_reduce_population_count(kd))

            offb, offd = stream_pass(row, fnc, (zero, zero))

            zero_hists()

            def mid_b(i, car):
                v = cand_b[pl.ds(i * _L, _L)]
                msk = (i * _L + lane) < offb
                plsc.addupdate_scatter(hist_b, [(v >> 8) & 0xFFF], ones,
                                       mask=msk)
                return car

            def mid_d(i, car):
                v = cand_d[pl.ds(i * _L, _L)]
                msk = (i * _L + lane) < offd
                plsc.addupdate_scatter(hist_d, [(v >> 8) & 0xFFF], ones,
                                       mask=msk)
                return car

            nvb = (jnp.max(offb) + _L - 1) // _L
            nvd = (jnp.max(offd) + _L - 1) // _L
            lax.fori_loop(0, nvb, mid_b, jnp.int32(0))
            lax.fori_loop(0, nvd, mid_d, jnp.int32(0))
            p1b, chb2 = scan_hist(hist_b, 4096 // _L, chib)
            p1d, chd2 = scan_hist(hist_d, 4096 // _L, chid)

            zero_hists()

            def low_b(i, car):
                v = cand_b[pl.ds(i * _L, _L)]
                msk = ((i * _L + lane) < offb) & (((v >> 8) & 0xFFF) == p1b)
                plsc.addupdate_scatter(hist_b, [v & 0xFF], ones, mask=msk)
                return car

            def low_d(i, car):
                v = cand_d[pl.ds(i * _L, _L)]
                msk = ((i * _L + lane) < offd) & (((v >> 8) & 0xFFF) == p1d)
                plsc.addupdate_scatter(hist_d, [v & 0xFF], ones, mask=msk)
                return car

            lax.fori_loop(0, nvb, low_b, jnp.int32(0))
            lax.fori_loop(0, nvd, low_d, jnp.int32(0))
            p2b, _ = scan_hist(hist_b, 256 // _L, chb2)
            p2d, _ = scan_hist(hist_d, 256 // _L, chd2)

            flag = jnp.maximum(c0, jnp.maximum(offb, offd))
        else:
            zero_hists()

            def fn1(xb, xd, sb, sd, car):
                plsc.addupdate_scatter(hist_b, [(sb >> 8) & 0xFFF], ones,
                                       mask=((sb >> 20) + 2048) == p0b)
                plsc.addupdate_scatter(hist_d, [(sd >> 8) & 0xFFF], ones,
                                       mask=((sd >> 20) + 2048) == p0d)
                return car

            stream_pass(row, fn1, jnp.int32(0))
            p1b, chb2 = scan_hist(hist_b, 4096 // _L, chib)
            p1d, chd2 = scan_hist(hist_d, 4096 // _L, chid)

            zero_hists()
            pre_b = ((p0b - 2048) << 12) + p1b
            pre_d = ((p0d - 2048) << 12) + p1d

            def fn2(xb, xd, sb, sd, car):
                plsc.addupdate_scatter(hist_b, [sb & 0xFF], ones,
                                       mask=(sb >> 8) == pre_b)
                plsc.addupdate_scatter(hist_d, [sd & 0xFF], ones,
                                       mask=(sd >> 8) == pre_d)
                return car

            stream_pass(row, fn2, jnp.int32(0))
            p2b, _ = scan_hist(hist_b, 256 // _L, chb2)
            p2d, _ = scan_hist(hist_d, 256 // _L, chd2)

        s_b = ((p0b - 2048) << 20) + (p1b << 8) + p2b
        s_d = ((p0d - 2048) << 20) + (p1d << 8) + p2d
        tfb = lax.bitcast_convert_type(_sortable(s_b), jnp.float32)
        tfd = lax.bitcast_convert_type(_sortable(s_d), jnp.float32)
        mfb = jnp.zeros((_L,), jnp.float32) + jnp.max(mb)
        mfd = jnp.zeros((_L,), jnp.float32) + jnp.max(md)

        cur = obuf[...]
        cur = jnp.where(lane == rr, tfb, cur)
        cur = jnp.where(lane == rpt + rr, tfd, cur)
        cur = jnp.where(lane == 2 * rpt + rr, mfb, cur)
        cur = jnp.where(lane == 3 * rpt + rr, mfd, cur)
        obuf[...] = cur
        return flag if collect else c0

    flag = lax.fori_loop(0, rpt, row_task, jnp.zeros((_L,), jnp.int32))
    fbuf[...] = flag
    pltpu.sync_copy(obuf, out_hbm.at[wid])
    pltpu.sync_copy(fbuf, flag_hbm.at[wid])
  return _sc_body


def _sc_stats(pb, pd, lik, nrows, collect):
    mesh = plsc.VectorSubcoreMesh(core_axis_name="c", subcore_axis_name="s",
                                  num_cores=2, num_subcores=16)
    f32 = jnp.float32
    return pl.kernel(
        _make_sc_body(nrows // _NTILES, collect),
        out_type=[jax.ShapeDtypeStruct((_NTILES, _L), f32),
                  jax.ShapeDtypeStruct((_NTILES, _L), jnp.int32)],
        mesh=mesh,
        compiler_params=pltpu.CompilerParams(needs_layout_passes=False),
        scratch_types=[
            pltpu.VMEM((_CHUNK,), f32), pltpu.VMEM((_CHUNK,), f32),
            pltpu.VMEM((_CHUNK,), f32), pltpu.VMEM((_CHUNK,), f32),
            pltpu.VMEM((_CHUNK,), f32), pltpu.VMEM((_CHUNK,), f32),
            pltpu.VMEM((4096,), jnp.int32), pltpu.VMEM((4096,), jnp.int32),
            pltpu.VMEM((_CAP,), jnp.int32), pltpu.VMEM((_CAP,), jnp.int32),
            pltpu.VMEM((_L,), f32), pltpu.VMEM((_L,), jnp.int32),
            pltpu.SemaphoreType.DMA, pltpu.SemaphoreType.DMA,
            pltpu.SemaphoreType.DMA, pltpu.SemaphoreType.DMA,
            pltpu.SemaphoreType.DMA, pltpu.SemaphoreType.DMA,
        ],
    )(pb, pd, lik)


def _tc_body(pb_ref, pd_ref, lik_ref, st_ref, out_ref):
    lik = lik_ref[...]
    st = st_ref[...]
    for src, p_ref in ((0, pb_ref), (1, pd_ref)):
        x = p_ref[...] + lik
        t_f = st[:, src:src + 1]
        m_f = st[:, 2 + src:3 + src]
        e = jnp.where(x >= t_f, jnp.exp(x - m_f), jnp.float32(0.0))
        denom = jnp.sum(e, axis=-1, keepdims=True)
        out_ref[src] = e * (jnp.float32(1.0) / denom)


def kernel(prior_bass_logits, prior_drums_logits, likelihood_logits, top_k):
    del top_k
    pb, pd, lik = prior_bass_logits, prior_drums_logits, likelihood_logits
    B, V = pb.shape
    pb1, pd1, lik1 = pb.reshape(-1), pd.reshape(-1), lik.reshape(-1)
    stats_fast, flags = _sc_stats(pb1, pd1, lik1, B, collect=True)
    ok = jnp.max(flags) <= _CAP
    stats = lax.cond(
        ok,
        lambda _: stats_fast,
        lambda _: _sc_stats(pb1, pd1, lik1, B, collect=False)[0],
        0)
    rpt = B // _NTILES
    st = stats[:, :4 * rpt].reshape(_NTILES, 4, rpt)
    st = st.transpose(0, 2, 1).reshape(B, 4)
    in_spec = pl.BlockSpec((_RB, V), lambda i: (i, 0))
    return pl.pallas_call(
        _tc_body,
        grid=(B // _RB,),
        in_specs=[in_spec, in_spec, in_spec,
                  pl.BlockSpec((_RB, 4), lambda i: (i, 0))],
        out_specs=pl.BlockSpec((2, _RB, V), lambda i: (0, i, 0)),
        out_shape=jax.ShapeDtypeStruct((2, B, V), jnp.float32),
    )(pb, pd, lik, st)


# speedup vs baseline: 1.4556x; 1.2619x over previous
import jax
import jax.numpy as jnp
from jax import lax
from jax.experimental import pallas as pl
from jax.experimental.pallas import tpu as pltpu
from jax.experimental.pallas import tpu_sc as plsc

_K = 256
_RB = 8
_V = 100000
_CW = 1664
_NCH = _V // _CW
_TAIL = _V - _NCH * _CW
_GR = 8
_NTILES = 32
_L = 16
_CAP = 2048


def _sortable(b):
    return b ^ ((b >> 31) & jnp.int32(0x7FFFFFFF))


def _make_sc_body(collect):
  def _sc_body(pb_hbm, pd_hbm, lik_hbm, tpb_hbm, tpd_hbm, tlik_hbm,
               out_hbm, flag_hbm,
               p0buf, p1buf, l0buf, l1buf, tpbuf, tlbuf, hist, cand, obuf,
               fbuf, sp0, sp1, sl0, sl1, stp, stl):
    wid = lax.axis_index("s") * 2 + lax.axis_index("c")
    src_is_b = (wid & 1) == 0
    grp = wid >> 1
    row0 = pl.multiple_of(grp * _GR, _GR)
    lane = lax.iota(jnp.int32, _L)
    ones = jnp.ones((_L,), jnp.int32)
    zero = jnp.zeros((_L,), jnp.int32)
    pslots = (p0buf, p1buf)
    lslots = (l0buf, l1buf)
    psems = (sp0, sp1)
    lsems = (sl0, sl1)

    def start(ci, slot):
        col = pl.multiple_of(ci * _CW, 128)
        sl = (pl.ds(row0, _GR), pl.ds(col, _CW))

        @pl.when(src_is_b)
        def _():
            pltpu.async_copy(pb_hbm.at[sl], pslots[slot], psems[slot])

        @pl.when(jnp.logical_not(src_is_b))
        def _():
            pltpu.async_copy(pd_hbm.at[sl], pslots[slot], psems[slot])

        pltpu.async_copy(lik_hbm.at[sl], lslots[slot], lsems[slot])

    def wait(slot):
        pltpu.make_async_copy(pb_hbm.at[pl.ds(0, _GR), pl.ds(0, _CW)],
                              pslots[slot], psems[slot]).wait()
        pltpu.make_async_copy(lik_hbm.at[pl.ds(0, _GR), pl.ds(0, _CW)],
                              lslots[slot], lsems[slot]).wait()

    def chunk_compute(slot, nv, fn, carries):
        pr, lr = pslots[slot], lslots[slot]
        out = []
        for r in range(_GR):
            def body(i, car, r=r, pr=pr, lr=lr):
                sl = pl.ds(i * _L, _L)
                x = pr[r, sl] + lr[r, sl]
                s = _sortable(lax.bitcast_convert_type(x, jnp.int32))
                return fn(r, x, s, car)
            out.append(plsc.parallel_loop(0, nv, 1, unroll=4,
                                          carry=carries[r])(body))
        return tuple(out)

    def tail_compute(fn, carries):
        out = []
        for r in range(_GR):
            def body(i, car, r=r):
                sl = pl.ds(i * _L, _L)
                x = tpbuf[r, sl] + tlbuf[r, sl]
                s = _sortable(lax.bitcast_convert_type(x, jnp.int32))
                return fn(r, x, s, car)
            out.append(plsc.parallel_loop(0, _TAIL // _L, 1, unroll=2,
                                          carry=carries[r])(body))
        return tuple(out)

    def stream_pass(fn, carries):
        start(0, 0)
        start(1, 1)

        def dbl(cc, cars):
            for slot in (0, 1):
                ci = 2 * cc + slot
                wait(slot)
                cars = chunk_compute(slot, _CW // _L, fn, cars)

                @pl.when(ci + 2 < _NCH)
                def _():
                    start(ci + 2, slot)
            return cars

        carries = lax.fori_loop(0, _NCH // 2, dbl, tuple(carries))
        tsl = pl.ds(row0, _GR)

        @pl.when(src_is_b)
        def _():
            pltpu.async_copy(tpb_hbm.at[tsl], tpbuf, stp)

        @pl.when(jnp.logical_not(src_is_b))
        def _():
            pltpu.async_copy(tpd_hbm.at[tsl], tpbuf, stp)

        pltpu.async_copy(tlik_hbm.at[tsl], tlbuf, stl)
        pltpu.make_async_copy(tpb_hbm.at[tsl], tpbuf, stp).wait()
        pltpu.make_async_copy(tlik_hbm.at[tsl], tlbuf, stl).wait()
        return tail_compute(fn, carries)

    def zero_hist_region(base, nvregs):
        @plsc.parallel_loop(0, nvregs, 1, unroll=8)
        def _(j):
            hist[pl.ds(base + j * _L, _L)] = zero

    def scan_hist(base, nvregs, running0):
        def body(j, car):
            running, p, c_hi, found = car
            boff = base + (nvregs - 1 - j) * _L
            v = hist[pl.ds(boff, _L)]
            rv = lax.rev(v, (0,))
            cs = plsc.cumsum(rv)
            tot = running + cs
            pred = tot >= _K
            npred = plsc.all_reduce_population_count(pred)
            f = plsc.all_reduce_ffs(pred)
            hit = (npred > 0) & jnp.logical_not(found)
            p_new = (nvregs - 1 - j) * _L + 15 - f
            c_hi_new = running + jnp.sum(jnp.where(lane < f, rv, zero))
            run_next = running + jnp.sum(v)
            return (run_next,
                    jnp.where(hit, p_new, p),
                    jnp.where(hit, c_hi_new, c_hi),
                    found | (npred > 0))

        init = (running0, zero, zero, jnp.zeros((_L,), jnp.bool_))
        _, p, c_hi, _ = plsc.parallel_loop(0, nvregs, 1, unroll=4,
                                           carry=init)(body)
        return p, c_hi

    for r in range(_GR):
        zero_hist_region(r * 4096, 4096 // _L)

    def fn0(r, x, s, car):
        plsc.addupdate_scatter(hist, [r * 4096 + ((s >> 20) + 2048)], ones)
        return jnp.maximum(car, x)

    ninf = jnp.full((_L,), -jnp.inf, jnp.float32)
    maxes = stream_pass(fn0, (ninf,) * _GR)

    p0 = []
    chi = []
    for r in range(_GR):
        p, c = scan_hist(r * 4096, 4096 // _L, zero)
        p0.append(p)
        chi.append(c)

    if collect:
        def fnc(r, x, s, car):
            kb = ((s >> 20) + 2048) == p0[r]
            pos = car + plsc.cumsum(kb.astype(jnp.int32)) - 1
            plsc.store_scatter(cand, [r * _CAP + pos], s,
                               mask=kb & (pos < _CAP))
            return car + plsc.all_reduce_population_count(kb)

        offs = stream_pass(fnc, (zero,) * _GR)

        flag = zero
        thr = []
        for r in range(_GR):
            flag = jnp.maximum(flag, offs[r])
            nv = (jnp.max(offs[r]) + _L - 1) // _L

            zero_hist_region(0, 4096 // _L)

            def mid(i, car, r=r):
                v = cand[pl.ds(r * _CAP + i * _L, _L)]
                msk = (i * _L + lane) < offs[r]
                plsc.addupdate_scatter(hist, [(v >> 8) & 0xFFF], ones,
                                       mask=msk)
                return car

            lax.fori_loop(0, nv, mid, jnp.int32(0))
            p1, c2 = scan_hist(0, 4096 // _L, chi[r])

            zero_hist_region(0, 256 // _L)

            def low(i, car, r=r, p1=p1):
                v = cand[pl.ds(r * _CAP + i * _L, _L)]
                msk = (((i * _L + lane) < offs[r])
                       & (((v >> 8) & 0xFFF) == p1))
                plsc.addupdate_scatter(hist, [v & 0xFF], ones, mask=msk)
                return car

            lax.fori_loop(0, nv, low, jnp.int32(0))
            p2, _ = scan_hist(0, 256 // _L, c2)
            thr.append(((p0[r] - 2048) << 20) + (p1 << 8) + p2)
        fbuf[...] = flag
    else:
        for r in range(_GR):
            zero_hist_region(r * 4096, 4096 // _L)

        def fn1(r, x, s, car):
            plsc.addupdate_scatter(hist, [r * 4096 + ((s >> 8) & 0xFFF)],
                                   ones, mask=((s >> 20) + 2048) == p0[r])
            return car

        stream_pass(fn1, (zero,) * _GR)
        p1 = []
        ch2 = []
        for r in range(_GR):
            p, c = scan_hist(r * 4096, 4096 // _L, chi[r])
            p1.append(p)
            ch2.append(c)

        for r in range(_GR):
            zero_hist_region(r * 4096, 256 // _L)

        def fn2(r, x, s, car):
            pre = ((p0[r] - 2048) << 12) + p1[r]
            plsc.addupdate_scatter(hist, [r * 4096 + (s & 0xFF)], ones,
                                   mask=(s >> 8) == pre)
            return car

        stream_pass(fn2, (zero,) * _GR)
        thr = []
        for r in range(_GR):
            p2, _ = scan_hist(r * 4096, 256 // _L, ch2[r])
            thr.append(((p0[r] - 2048) << 20) + (p1[r] << 8) + p2)
        fbuf[...] = zero

    cur = obuf[...]
    for r in range(_GR):
        tf = lax.bitcast_convert_type(_sortable(thr[r]), jnp.float32)
        mf = jnp.zeros((_L,), jnp.float32) + jnp.max(maxes[r])
        cur = jnp.where(lane == r, tf, cur)
        cur = jnp.where(lane == _GR + r, mf, cur)
    obuf[...] = cur
    pltpu.sync_copy(obuf, out_hbm.at[wid])
    pltpu.sync_copy(fbuf, flag_hbm.at[wid])
  return _sc_body


def _sc_stats(pb, pd, lik, tpb, tpd, tlik, collect):
    mesh = plsc.VectorSubcoreMesh(core_axis_name="c", subcore_axis_name="s",
                                  num_cores=2, num_subcores=16)
    f32 = jnp.float32
    return pl.kernel(
        _make_sc_body(collect),
        out_type=[jax.ShapeDtypeStruct((_NTILES, _L), f32),
                  jax.ShapeDtypeStruct((_NTILES, _L), jnp.int32)],
        mesh=mesh,
        compiler_params=pltpu.CompilerParams(needs_layout_passes=False),
        scratch_types=[
            pltpu.VMEM((_GR, _CW), f32), pltpu.VMEM((_GR, _CW), f32),
            pltpu.VMEM((_GR, _CW), f32), pltpu.VMEM((_GR, _CW), f32),
            pltpu.VMEM((_GR, _TAIL), f32), pltpu.VMEM((_GR, _TAIL), f32),
            pltpu.VMEM((_GR * 4096,), jnp.int32),
            pltpu.VMEM((_GR * _CAP,), jnp.int32),
            pltpu.VMEM((_L,), f32), pltpu.VMEM((_L,), jnp.int32),
            pltpu.SemaphoreType.DMA, pltpu.SemaphoreType.DMA,
            pltpu.SemaphoreType.DMA, pltpu.SemaphoreType.DMA,
            pltpu.SemaphoreType.DMA, pltpu.SemaphoreType.DMA,
        ],
    )(pb, pd, lik, tpb, tpd, tlik)


def _tc_body(pb_ref, pd_ref, lik_ref, st_ref, out_ref):
    lik = lik_ref[...]
    st = st_ref[...]
    for src, p_ref in ((0, pb_ref), (1, pd_ref)):
        x = p_ref[...] + lik
        t_f = st[:, src:src + 1]
        m_f = st[:, 2 + src:3 + src]
        e = jnp.where(x >= t_f, jnp.exp(x - m_f), jnp.float32(0.0))
        denom = jnp.sum(e, axis=-1, keepdims=True)
        out_ref[src] = e * (jnp.float32(1.0) / denom)


def kernel(prior_bass_logits, prior_drums_logits, likelihood_logits, top_k):
    del top_k
    pb, pd, lik = prior_bass_logits, prior_drums_logits, likelihood_logits
    B, V = pb.shape
    tcol = _NCH * _CW
    tpb = pb[:, tcol:]
    tpd = pd[:, tcol:]
    tlik = lik[:, tcol:]
    stats_fast, flags = _sc_stats(pb, pd, lik, tpb, tpd, tlik, collect=True)
    ok = jnp.max(flags) <= _CAP
    stats = lax.cond(
        ok,
        lambda _: stats_fast,
        lambda _: _sc_stats(pb, pd, lik, tpb, tpd, tlik, collect=False)[0],
        0)
    s4 = stats.reshape(16, 2, 2, _GR)
    st = s4.transpose(0, 3, 2, 1).reshape(B, 4)
    in_spec = pl.BlockSpec((_RB, V), lambda i: (i, 0))
    return pl.pallas_call(
        _tc_body,
        grid=(B // _RB,),
        in_specs=[in_spec, in_spec, in_spec,
                  pl.BlockSpec((_RB, 4), lambda i: (i, 0))],
        out_specs=pl.BlockSpec((2, _RB, V), lambda i: (0, i, 0)),
        out_shape=jax.ShapeDtypeStruct((2, B, V), jnp.float32),
    )(pb, pd, lik, st)
